# Initial kernel scaffold; baseline (speedup 1.0000x reference)
import jax
import jax.numpy as jnp
from jax.experimental import pallas as pl


def kernel(x, embeddings):
    raise NotImplementedError("write your pallas kernel here")


# speedup vs baseline: 1.1918x; 1.1918x over previous
import functools

import jax
import jax.numpy as jnp
from jax import lax
from jax.experimental import pallas as pl
from jax.experimental.pallas import tpu as pltpu
from jax.experimental.pallas import tpu_sc as plsc

N_EMB = 8192
D = 32
N_TOK = 32768
COMMIT = 0.25

NUM_CORES = 2
NUM_SUBCORES = 16
NW = NUM_CORES * NUM_SUBCORES
B_PER_W = N_TOK // NW
CHUNK = 128
N_CHUNK = B_PER_W // CHUNK
HIST_W = 16


def _sc_call(table, idx_w, ones_hb, zeros_hb):
    mesh = plsc.VectorSubcoreMesh(
        core_axis_name="c", subcore_axis_name="s",
        num_cores=NUM_CORES, num_subcores=NUM_SUBCORES)

    @functools.partial(
        pl.kernel,
        mesh=mesh,
        compiler_params=pltpu.CompilerParams(use_tc_tiling_on_sc=False),
        out_type=[
            jax.ShapeDtypeStruct((NW, N_CHUNK, CHUNK, D), jnp.float32),
            jax.ShapeDtypeStruct((NUM_CORES, N_EMB, HIST_W), jnp.float32),
        ],
        scratch_types=[
            pltpu.VMEM((N_CHUNK, CHUNK), jnp.int32),
            pltpu.VMEM((N_CHUNK, CHUNK, D), jnp.float32),
            pltpu.VMEM((CHUNK, HIST_W), jnp.float32),
            pltpu.VMEM_SHARED((N_EMB, HIST_W), jnp.float32),
            pltpu.SemaphoreType.DMA,
        ],
    )
    def k(table_hbm, idx_hbm, ones_hbm, zeros_hbm, q_hbm, cnt_hbm,
          idx_v, rows_v, ones_v, hist_sh, sem):
        c = lax.axis_index("c")
        s = lax.axis_index("s")
        wid = s * NUM_CORES + c
        pltpu.sync_copy(idx_hbm.at[wid], idx_v)
        pltpu.sync_copy(ones_hbm, ones_v)

        @pl.when(s == 0)
        def _():
            pltpu.sync_copy(zeros_hbm, hist_sh)

        for j in range(N_CHUNK):
            pltpu.async_copy(table_hbm.at[idx_v.at[j]], rows_v.at[j], sem).wait()
        pltpu.sync_copy(rows_v, q_hbm.at[wid])

        plsc.subcore_barrier()
        for j in range(N_CHUNK):
            pltpu.sync_copy(ones_v, hist_sh.at[idx_v.at[j]], add=True)
        plsc.subcore_barrier()

        @pl.when(s == 0)
        def _():
            pltpu.sync_copy(hist_sh, cnt_hbm.at[c])

    return k(table, idx_w, ones_hb, zeros_hb)


def _finalize_body(cnt_ref, q_ref, x_ref, loss_ref, perp_ref, avg_ref):
    cnt = cnt_ref[0] + cnt_ref[1]
    avg = cnt * (1.0 / N_TOK)
    avg_ref[...] = avg
    a0 = avg[:, 0:1]
    ent = jnp.sum(a0 * jnp.log(a0 + 1e-10))
    perp_ref[...] = jnp.full((8, 128), jnp.exp(-ent), jnp.float32)
    diff = q_ref[...] - x_ref[...]
    scale = (1.0 + COMMIT) / (N_TOK * D)
    loss_ref[...] = jnp.full((8, 128), jnp.sum(diff * diff) * scale,
                             jnp.float32)


def _finalize_call(cnt2, qflat, flat):
    return pl.pallas_call(
        _finalize_body,
        in_specs=[
            pl.BlockSpec((NUM_CORES, N_EMB, HIST_W), lambda: (0, 0, 0)),
            pl.BlockSpec((N_TOK, D), lambda: (0, 0)),
            pl.BlockSpec((N_TOK, D), lambda: (0, 0)),
        ],
        out_specs=[
            pl.BlockSpec((8, 128), lambda: (0, 0)),
            pl.BlockSpec((8, 128), lambda: (0, 0)),
            pl.BlockSpec((N_EMB, HIST_W), lambda: (0, 0)),
        ],
        out_shape=[
            jax.ShapeDtypeStruct((8, 128), jnp.float32),
            jax.ShapeDtypeStruct((8, 128), jnp.float32),
            jax.ShapeDtypeStruct((N_EMB, HIST_W), jnp.float32),
        ],
    )(cnt2, qflat, flat)


def kernel(x, embeddings):
    xf, ef = lax.optimization_barrier((x, embeddings))
    fl = xf.reshape((-1, D))
    dist = (jnp.sum(fl * fl, 1, keepdims=True)
            + jnp.sum(ef * ef, 0, keepdims=True)
            - 2.0 * fl @ ef)
    enc_idx = lax.optimization_barrier(jnp.argmax(-dist, 1))
    flat = x.reshape(N_TOK, D)
    idx_w = enc_idx.reshape(NW, N_CHUNK, CHUNK)
    table = lax.optimization_barrier(embeddings.T)
    ones_hb = jnp.ones((CHUNK, HIST_W), jnp.float32)
    zeros_hb = jnp.zeros((N_EMB, HIST_W), jnp.float32)
    q4, cnt2 = _sc_call(table, idx_w, ones_hb, zeros_hb)
    qflat = q4.reshape(N_TOK, D)
    loss8, perp8, avg16 = _finalize_call(cnt2, qflat, flat)
    quantized_st = qflat.reshape(x.shape)
    return (quantized_st, loss8[0, 0], perp8[0, 0],
            enc_idx.reshape(x.shape[:-1]), avg16[:, 0])
